# Initial kernel scaffold; baseline (speedup 1.0000x reference)
import jax
import jax.numpy as jnp
from jax.experimental import pallas as pl


def kernel(x, pos, edges_0, batch, W, b, attn, W_rbf, w_read, b_read):
    raise NotImplementedError("write your pallas kernel here")


# speedup vs baseline: 2.3399x; 2.3399x over previous
import functools

import jax
import jax.numpy as jnp
from jax import lax
from jax.experimental import pallas as pl
from jax.experimental.pallas import tpu as pltpu
from jax.experimental.pallas import tpu_sc as plsc

T_W = 256
WIN = 128
NW = 32


def _sc_gather2(table, src, dst):
    E = src.shape[0]
    mesh = plsc.VectorSubcoreMesh(core_axis_name="c", subcore_axis_name="s")
    grid_e = E // WIN
    per_w = grid_e // NW

    @functools.partial(
        pl.kernel,
        out_type=(jax.ShapeDtypeStruct((E, T_W), jnp.float32),
                  jax.ShapeDtypeStruct((E, T_W), jnp.float32)),
        mesh=mesh,
    )
    def k(t_hbm, s_hbm, d_hbm, os_hbm, od_hbm):
        def body(i_vmem, o_vmem):
            pltpu.sync_copy(t_hbm.at[i_vmem.at[0]], o_vmem)

        for idx_hbm, out_hbm in ((s_hbm, os_hbm), (d_hbm, od_hbm)):
            pltpu.emit_pipeline(
                body,
                grid=(NW, per_w),
                in_specs=[
                    pl.BlockSpec((1, WIN), lambda w, i: (0, w * per_w + i)),
                ],
                out_specs=[
                    pl.BlockSpec((WIN, T_W), lambda w, i: (w * per_w + i, 0)),
                ],
                core_axis_name=("c", "s"),
                dimension_semantics=(pltpu.PARALLEL, pltpu.ARBITRARY),
            )(idx_hbm, out_hbm)

    return k(table, src.reshape(1, E), dst.reshape(1, E))


def kernel(x, pos, edges_0, batch, W, b, attn, W_rbf, w_read, b_read):
    N, D_IN = x.shape
    DIM = W.shape[0]
    E = edges_0.shape[0]
    G = 64
    NG = W_rbf.shape[1]

    src = edges_0[:, 0]
    dst = edges_0[:, 1]
    E_pad = ((E + WIN * NW - 1) // (WIN * NW)) * (WIN * NW)
    pad = E_pad - E
    src_p = jnp.concatenate([src, jnp.zeros((pad,), jnp.int32)])
    dst_p = jnp.concatenate([dst, jnp.zeros((pad,), jnp.int32)])

    h = x @ W.T + b
    table = jnp.concatenate(
        [h, pos, batch.astype(jnp.float32)[:, None],
         jnp.zeros((N, T_W - DIM - 4), jnp.float32)], axis=1)
    assert table.shape[1] == T_W

    Ts, Td = _sc_gather2(table, src_p, dst_p)
    Ts = Ts[:E]
    Td = Td[:E]

    hs = Ts[:, :DIM] + Td[:, :DIM]
    e = jax.nn.elu(hs)
    vec = Td[:, DIM:DIM + 3] - Ts[:, DIM:DIM + 3]
    distance = jnp.sqrt(jnp.sum(vec * vec, axis=1, keepdims=True))
    distance = jnp.maximum(distance, 0.1)
    offsets = jnp.linspace(0.0, 5.0, NG)
    gap = offsets[1] - offsets[0]
    coeff = -0.5 / (gap ** 2)
    rbf = jnp.exp(coeff * (distance - offsets[None, :]) ** 2)
    dist_emd = rbf @ W_rbf.T
    e = e * dist_emd
    atom_repr = jax.nn.elu(e * attn)
    batch_e = Ts[:, DIM + 3].astype(jnp.int32)
    score = jax.nn.sigmoid(atom_repr @ w_read + b_read)
    out_sum = jax.ops.segment_sum(atom_repr * score, batch_e, num_segments=G)
    out_max = jax.ops.segment_max(atom_repr, batch_e, num_segments=G)
    return jnp.concatenate([out_sum, out_max], axis=1)


# speedup vs baseline: 2.8923x; 1.2361x over previous
import functools

import jax
import jax.numpy as jnp
from jax import lax
from jax.experimental import pallas as pl
from jax.experimental.pallas import tpu as pltpu
from jax.experimental.pallas import tpu_sc as plsc

T_W = 256
WIN = 128
NW = 32
EB = 2048
G = 64
NEG = -3e38


def _table_kernel(x, pos, batch, W, b):
    N, D_IN = x.shape
    DIM = W.shape[0]
    BLK = 1000
    grid = N // BLK

    def body(x_ref, pos_ref, bat_ref, w_ref, b_ref, t_ref):
        h = jnp.dot(x_ref[...], w_ref[...].T,
                    preferred_element_type=jnp.float32) + b_ref[...]
        bf = bat_ref[...].astype(jnp.float32)
        z = jnp.zeros((BLK, T_W - DIM - 4), jnp.float32)
        t_ref[...] = jnp.concatenate([h, pos_ref[...], bf, z], axis=1)

    return pl.pallas_call(
        body,
        grid=(grid,),
        in_specs=[
            pl.BlockSpec((BLK, D_IN), lambda i: (i, 0)),
            pl.BlockSpec((BLK, 3), lambda i: (i, 0)),
            pl.BlockSpec((BLK, 1), lambda i: (i, 0)),
            pl.BlockSpec((DIM, D_IN), lambda i: (0, 0)),
            pl.BlockSpec((1, DIM), lambda i: (0, 0)),
        ],
        out_specs=pl.BlockSpec((BLK, T_W), lambda i: (i, 0)),
        out_shape=jax.ShapeDtypeStruct((N, T_W), jnp.float32),
    )(x, pos, batch[:, None], W, b[None, :])


def _sc_gather(table, idx_all):
    M = idx_all.shape[0]
    mesh = plsc.VectorSubcoreMesh(core_axis_name="c", subcore_axis_name="s")
    per_w = M // WIN // NW

    @functools.partial(
        pl.kernel,
        out_type=jax.ShapeDtypeStruct((M, T_W), jnp.float32),
        mesh=mesh,
    )
    def k(t_hbm, i_hbm, o_hbm):
        def body(i_vmem, o_vmem):
            pltpu.sync_copy(t_hbm.at[i_vmem.at[0]], o_vmem)

        pltpu.emit_pipeline(
            body,
            grid=(NW, per_w),
            in_specs=[pl.BlockSpec((1, WIN), lambda w, i: (0, w * per_w + i))],
            out_specs=[pl.BlockSpec((WIN, T_W), lambda w, i: (w * per_w + i, 0))],
            core_axis_name=("c", "s"),
            dimension_semantics=(pltpu.PARALLEL, pltpu.ARBITRARY),
        )(i_hbm, o_hbm)

    return k(table, idx_all.reshape(1, M))


def _edge_kernel(tstd, w_rbf_t, attn, w_read_t, b_read, e_real, n_blocks):
    E_pad = tstd.shape[0] // 2
    DIM = attn.shape[1]
    NG = w_rbf_t.shape[0]
    offs = [5.0 * k / (NG - 1) for k in range(NG)]
    gap = offs[1] - offs[0]
    coeff = -0.5 / (gap * gap)

    def body(ts_ref, td_ref, wr_ref, at_ref, wread_ref, bread_ref,
             sum_ref, atom_ref, be_ref):
        i = pl.program_id(0)
        ts = ts_ref[...]
        td = td_ref[...]
        hs = ts[:, :DIM] + td[:, :DIM]
        e1 = jnp.where(hs > 0, hs, jnp.exp(hs) - 1.0)
        d2 = jnp.zeros((EB, 1), jnp.float32)
        for c in range(3):
            dv = ts[:, DIM + c:DIM + c + 1] - td[:, DIM + c:DIM + c + 1]
            d2 = d2 + dv * dv
        dist = jnp.maximum(jnp.sqrt(d2), 0.1)
        emd = jnp.zeros((EB, DIM), jnp.float32)
        for k in range(NG):
            rk = jnp.exp(coeff * (dist - offs[k]) ** 2)
            emd = emd + rk * wr_ref[k:k + 1, :]
        e2 = e1 * emd
        z = e2 * at_ref[...]
        atom = jnp.where(z > 0, z, jnp.exp(z) - 1.0)
        logit = jnp.sum(atom * wread_ref[...], axis=1, keepdims=True)
        score = jax.nn.sigmoid(logit + bread_ref[...])
        rows = i * EB + lax.broadcasted_iota(jnp.int32, (EB, 1), 0)
        valid = rows < e_real
        y = jnp.where(valid, atom * score, 0.0)
        bf = ts[:, DIM + 3:DIM + 4]
        giota = lax.broadcasted_iota(jnp.int32, (1, G), 1).astype(jnp.float32)
        onehot = (bf == giota)
        part = lax.dot_general(onehot.astype(jnp.float32), y,
                               (((0,), (0,)), ((), ())),
                               preferred_element_type=jnp.float32)

        @pl.when(i == 0)
        def _():
            sum_ref[...] = jnp.zeros_like(sum_ref)

        sum_ref[...] += part
        atom_ref[...] = jnp.where(valid, atom, NEG)
        be_ref[...] = bf.astype(jnp.int32).reshape(1, EB)

    return pl.pallas_call(
        body,
        grid=(n_blocks,),
        in_specs=[
            pl.BlockSpec((EB, T_W), lambda i: (i, 0)),
            pl.BlockSpec((EB, T_W), lambda i, nb=n_blocks: (nb + i, 0)),
            pl.BlockSpec((NG, DIM), lambda i: (0, 0)),
            pl.BlockSpec((1, DIM), lambda i: (0, 0)),
            pl.BlockSpec((1, DIM), lambda i: (0, 0)),
            pl.BlockSpec((1, 1), lambda i: (0, 0)),
        ],
        out_specs=[
            pl.BlockSpec((G, DIM), lambda i: (0, 0)),
            pl.BlockSpec((EB, DIM), lambda i: (i, 0)),
            pl.BlockSpec((1, EB), lambda i: (0, i)),
        ],
        out_shape=[
            jax.ShapeDtypeStruct((G, DIM), jnp.float32),
            jax.ShapeDtypeStruct((E_pad, DIM), jnp.float32),
            jax.ShapeDtypeStruct((1, E_pad), jnp.int32),
        ],
        compiler_params=pltpu.CompilerParams(
            dimension_semantics=("arbitrary",)),
    )(tstd, tstd, w_rbf_t, attn, w_read_t, b_read)


def kernel(x, pos, edges_0, batch, W, b, attn, W_rbf, w_read, b_read):
    N, D_IN = x.shape
    DIM = W.shape[0]
    E = edges_0.shape[0]

    src = edges_0[:, 0]
    dst = edges_0[:, 1]
    unit = WIN * NW
    E_pad = ((E + unit - 1) // unit) * unit
    pad = E_pad - E
    zpad = jnp.zeros((pad,), jnp.int32)
    idx_all = jnp.concatenate([src, zpad, dst, zpad])

    table = _table_kernel(x, pos, batch, W, b)
    tstd = _sc_gather(table, idx_all)

    out_sum, atom, batch_e = _edge_kernel(
        tstd, W_rbf.T, attn, w_read.T, b_read[None, :], E, E_pad // EB)

    out_max = jax.ops.segment_max(atom, batch_e.reshape(-1), num_segments=G)
    return jnp.concatenate([out_sum, out_max], axis=1)


# speedup vs baseline: 3.0724x; 1.0623x over previous
import functools

import jax
import jax.numpy as jnp
from jax import lax
from jax.experimental import pallas as pl
from jax.experimental.pallas import tpu as pltpu
from jax.experimental.pallas import tpu_sc as plsc

T_W = 128
WIN = 128
NW = 32
EB = 2048
G = 64
NEG = -3e38


def _table_kernel(x, pos, batch, W, b):
    N, D_IN = x.shape
    DIM = W.shape[0]
    BLK = 1000
    grid = N // BLK

    def body(x_ref, pos_ref, bat_ref, w_ref, b_ref, t_ref):
        h = jnp.dot(x_ref[...], w_ref[...].T,
                    preferred_element_type=jnp.float32) + b_ref[...]
        hb = lax.bitcast_convert_type(h, jnp.int32) + 0x8000
        hi = jnp.bitwise_and(hb[:, :DIM // 2], jnp.int32(-65536))
        lo = jnp.bitwise_and(jnp.right_shift(hb[:, DIM // 2:], 16),
                             jnp.int32(0xFFFF))
        packed = jnp.bitwise_or(hi, lo)
        posb = lax.bitcast_convert_type(pos_ref[...], jnp.int32)
        z = jnp.zeros((BLK, T_W - DIM // 2 - 4), jnp.int32)
        t_ref[...] = jnp.concatenate([packed, posb, bat_ref[...], z], axis=1)

    return pl.pallas_call(
        body,
        grid=(grid,),
        in_specs=[
            pl.BlockSpec((BLK, D_IN), lambda i: (i, 0)),
            pl.BlockSpec((BLK, 3), lambda i: (i, 0)),
            pl.BlockSpec((BLK, 1), lambda i: (i, 0)),
            pl.BlockSpec((DIM, D_IN), lambda i: (0, 0)),
            pl.BlockSpec((1, DIM), lambda i: (0, 0)),
        ],
        out_specs=pl.BlockSpec((BLK, T_W), lambda i: (i, 0)),
        out_shape=jax.ShapeDtypeStruct((N, T_W), jnp.int32),
    )(x, pos, batch[:, None], W, b[None, :])


def _sc_gather(table, idx_all):
    M = idx_all.shape[0]
    mesh = plsc.VectorSubcoreMesh(core_axis_name="c", subcore_axis_name="s")
    per_w = M // WIN // NW

    @functools.partial(
        pl.kernel,
        out_type=jax.ShapeDtypeStruct((M, T_W), jnp.int32),
        mesh=mesh,
    )
    def k(t_hbm, i_hbm, o_hbm):
        def body(i_vmem, o_vmem):
            pltpu.sync_copy(t_hbm.at[i_vmem.at[0]], o_vmem)

        pltpu.emit_pipeline(
            body,
            grid=(NW, per_w),
            in_specs=[pl.BlockSpec((1, WIN), lambda w, i: (0, w * per_w + i))],
            out_specs=[pl.BlockSpec((WIN, T_W), lambda w, i: (w * per_w + i, 0))],
            core_axis_name=("c", "s"),
            dimension_semantics=(pltpu.PARALLEL, pltpu.ARBITRARY),
        )(i_hbm, o_hbm)

    return k(table, idx_all.reshape(1, M))


def _edge_kernel(tstd, w_rbf_t, attn, w_read_t, b_read, e_real, n_blocks):
    E_pad = tstd.shape[0] // 2
    DIM = attn.shape[1]
    NG = w_rbf_t.shape[0]
    offs = [5.0 * k / (NG - 1) for k in range(NG)]
    gap = offs[1] - offs[0]
    coeff = -0.5 / (gap * gap)

    def body(ts_ref, td_ref, wr_ref, at_ref, wread_ref, bread_ref,
             sum_ref, atom_ref, be_ref):
        i = pl.program_id(0)
        ts = ts_ref[...]
        td = td_ref[...]
        H2 = DIM // 2

        def unpack_h(t):
            w = t[:, :H2]
            h1 = lax.bitcast_convert_type(
                jnp.bitwise_and(w, jnp.int32(-65536)), jnp.float32)
            h2 = lax.bitcast_convert_type(
                jnp.left_shift(w, 16), jnp.float32)
            return jnp.concatenate([h1, h2], axis=1)

        hs = unpack_h(ts) + unpack_h(td)
        e1 = jnp.where(hs > 0, hs, jnp.exp(hs) - 1.0)
        d2 = jnp.zeros((EB, 1), jnp.float32)
        for c in range(3):
            dv = lax.bitcast_convert_type(
                ts[:, H2 + c:H2 + c + 1], jnp.float32) - \
                lax.bitcast_convert_type(
                td[:, H2 + c:H2 + c + 1], jnp.float32)
            d2 = d2 + dv * dv
        dist = jnp.maximum(jnp.sqrt(d2), 0.1)
        emd = jnp.zeros((EB, DIM), jnp.float32)
        for k in range(NG):
            rk = jnp.exp(coeff * (dist - offs[k]) ** 2)
            emd = emd + rk * wr_ref[k:k + 1, :]
        e2 = e1 * emd
        z = e2 * at_ref[...]
        atom = jnp.where(z > 0, z, jnp.exp(z) - 1.0)
        logit = jnp.sum(atom * wread_ref[...], axis=1, keepdims=True)
        score = jax.nn.sigmoid(logit + bread_ref[...])
        rows = i * EB + lax.broadcasted_iota(jnp.int32, (EB, 1), 0)
        valid = rows < e_real
        y = jnp.where(valid, atom * score, 0.0)
        bi = ts[:, H2 + 3:H2 + 4]
        giota = lax.broadcasted_iota(jnp.int32, (1, G), 1)
        onehot = (bi == giota)
        part = lax.dot_general(onehot.astype(jnp.float32), y,
                               (((0,), (0,)), ((), ())),
                               preferred_element_type=jnp.float32)

        @pl.when(i == 0)
        def _():
            sum_ref[...] = jnp.zeros_like(sum_ref)

        sum_ref[...] += part
        atom_ref[...] = jnp.where(valid, atom, NEG)
        be_ref[...] = bi.reshape(1, EB)

    return pl.pallas_call(
        body,
        grid=(n_blocks,),
        in_specs=[
            pl.BlockSpec((EB, T_W), lambda i: (i, 0)),
            pl.BlockSpec((EB, T_W), lambda i, nb=n_blocks: (nb + i, 0)),
            pl.BlockSpec((NG, DIM), lambda i: (0, 0)),
            pl.BlockSpec((1, DIM), lambda i: (0, 0)),
            pl.BlockSpec((1, DIM), lambda i: (0, 0)),
            pl.BlockSpec((1, 1), lambda i: (0, 0)),
        ],
        out_specs=[
            pl.BlockSpec((G, DIM), lambda i: (0, 0)),
            pl.BlockSpec((EB, DIM), lambda i: (i, 0)),
            pl.BlockSpec((1, EB), lambda i: (0, i)),
        ],
        out_shape=[
            jax.ShapeDtypeStruct((G, DIM), jnp.float32),
            jax.ShapeDtypeStruct((E_pad, DIM), jnp.float32),
            jax.ShapeDtypeStruct((1, E_pad), jnp.int32),
        ],
        compiler_params=pltpu.CompilerParams(
            dimension_semantics=("arbitrary",)),
    )(tstd, tstd, w_rbf_t, attn, w_read_t, b_read)


def kernel(x, pos, edges_0, batch, W, b, attn, W_rbf, w_read, b_read):
    N, D_IN = x.shape
    DIM = W.shape[0]
    E = edges_0.shape[0]

    src = edges_0[:, 0]
    dst = edges_0[:, 1]
    unit = WIN * NW
    E_pad = ((E + unit - 1) // unit) * unit
    pad = E_pad - E
    zpad = jnp.zeros((pad,), jnp.int32)
    idx_all = jnp.concatenate([src, zpad, dst, zpad])

    table = _table_kernel(x, pos, batch, W, b)
    tstd = _sc_gather(table, idx_all)

    out_sum, atom, batch_e = _edge_kernel(
        tstd, W_rbf.T, attn, w_read.T, b_read[None, :], E, E_pad // EB)

    out_max = jax.ops.segment_max(atom, batch_e.reshape(-1), num_segments=G)
    return jnp.concatenate([out_sum, out_max], axis=1)
